# baseline (speedup 1.0000x reference)
import functools

import jax
import jax.numpy as jnp
from jax import lax
from jax.experimental import pallas as pl
from jax.experimental.pallas import tpu as pltpu
from jax.experimental.pallas import tpu_sc as plsc

NC = 2
NS = 16
NW = NC * NS
LNS = 16

_MESH = plsc.VectorSubcoreMesh(core_axis_name="c", subcore_axis_name="s")
_DN = (((1,), (1,)), ((), ()))


def _iota16():
    return lax.broadcasted_iota(jnp.int32, (LNS,), 0)


def _proj(x, W_AE, b_AE, Wl, bl, Wr, br):
    n, d = x.shape
    hd = Wl.shape[0]
    BR = 1000

    def pack_half_rows(v):
        b16 = v.astype(jnp.bfloat16)
        lo = lax.bitcast_convert_type(b16[:, :hd // 2],
                                      jnp.uint16).astype(jnp.uint32)
        hi = lax.bitcast_convert_type(b16[:, hd // 2:],
                                      jnp.uint16).astype(jnp.uint32)
        return lax.bitcast_convert_type(lo | (hi << 16), jnp.int32)

    def body(x_ref, wae_ref, bae_ref, wl_ref, bl_ref, wr_ref, br_ref,
             inp_ref, xl_ref, xlb_ref, xrb_ref):
        xv = x_ref[...]
        inp = lax.dot_general(xv, wae_ref[...], _DN,
                              preferred_element_type=jnp.float32) + bae_ref[...]
        inp_ref[...] = inp
        xlv = lax.dot_general(inp, wl_ref[...], _DN,
                              preferred_element_type=jnp.float32) + bl_ref[...]
        xl_ref[...] = xlv
        xlb_ref[...] = pack_half_rows(xlv)
        xrv = lax.dot_general(inp, wr_ref[...], _DN,
                              preferred_element_type=jnp.float32) + br_ref[...]
        xrb_ref[...] = pack_half_rows(xrv)

    return pl.pallas_call(
        body,
        grid=(n // BR,),
        in_specs=[
            pl.BlockSpec((BR, d), lambda i: (i, 0)),
            pl.BlockSpec((d, d), lambda i: (0, 0)),
            pl.BlockSpec((1, d), lambda i: (0, 0)),
            pl.BlockSpec((hd, d), lambda i: (0, 0)),
            pl.BlockSpec((1, hd), lambda i: (0, 0)),
            pl.BlockSpec((hd, d), lambda i: (0, 0)),
            pl.BlockSpec((1, hd), lambda i: (0, 0)),
        ],
        out_specs=[
            pl.BlockSpec((BR, d), lambda i: (i, 0)),
            pl.BlockSpec((BR, hd), lambda i: (i, 0)),
            pl.BlockSpec((BR, hd // 2), lambda i: (i, 0)),
            pl.BlockSpec((BR, hd // 2), lambda i: (i, 0)),
        ],
        out_shape=[
            jax.ShapeDtypeStruct((n, d), jnp.float32),
            jax.ShapeDtypeStruct((n, hd), jnp.float32),
            jax.ShapeDtypeStruct((n, hd // 2), jnp.int32),
            jax.ShapeDtypeStruct((n, hd // 2), jnp.int32),
        ],
    )(x, W_AE, b_AE.reshape(1, d), Wl, bl.reshape(1, hd), Wr, br.reshape(1, hd))


def _edge_logits(xlb, xrb, src, dstg, dsts, att_perm, e_pad, ew, ce, npad):
    hdw = xlb.shape[1]
    hd = hdw * 2
    nch = ew // ce
    nj = hd // (2 * LNS)
    dl = npad * 4

    blkc = 36
    nblk = nch // blkc
    ib = blkc * ce
    nbuf = 4

    @functools.partial(
        pl.kernel,
        compiler_params=pltpu.CompilerParams(needs_layout_passes=False),
        out_type=[
            jax.ShapeDtypeStruct((e_pad * 4,), jnp.float32),
            jax.ShapeDtypeStruct((NW, dl), jnp.float32),
        ],
        mesh=_MESH,
        scratch_types=[
            pltpu.VMEM((hdw,), jnp.int32),
            pltpu.VMEM((ib,), jnp.int32),
            pltpu.VMEM((ib,), jnp.int32),
            pltpu.VMEM((ib,), jnp.int32),
            [pltpu.VMEM((ce, hdw), jnp.int32) for _ in range(nbuf)],
            [pltpu.VMEM((ce, hdw), jnp.int32) for _ in range(nbuf)],
            pltpu.VMEM((ce * 4 * LNS,), jnp.float32),
            pltpu.VMEM((ib * 4,), jnp.float32),
            pltpu.VMEM((dl,), jnp.float32),
            [pltpu.SemaphoreType.DMA for _ in range(nbuf)],
            [pltpu.SemaphoreType.DMA for _ in range(nbuf)],
        ],
    )
    def k(xl_hbm, xr_hbm, src_hbm, dstg_hbm, dsts_hbm, att_hbm,
          pexp_hbm, dpart_hbm,
          attv, srcb, dstb, dstsb, xlrows, xrrows, accb, pexpb, dloc,
          seml, semr):
        c = lax.axis_index("c")
        s = lax.axis_index("s")
        wid = s * NC + c
        base = wid * ew
        iota = _iota16()
        h4 = jnp.minimum(iota, 3)
        m4 = iota < 4

        zv = jnp.zeros((LNS,), jnp.float32)

        @pl.loop(0, dl // LNS)
        def _(t):
            dloc[pl.ds(t * LNS, LNS)] = zv
        pltpu.sync_copy(att_hbm, attv)
        atts = [plsc.bitcast(attv[pl.ds(j * LNS, LNS)], jnp.bfloat16)
                for j in range(nj)]

        def issue(r, b):
            pltpu.async_copy(xl_hbm.at[srcb.at[pl.ds(r * ce, ce)]],
                             xlrows[b], seml[b])
            pltpu.async_copy(xr_hbm.at[dstb.at[pl.ds(r * ce, ce)]],
                             xrrows[b], semr[b])

        def wait_rows(b):
            pltpu.make_async_copy(xl_hbm.at[srcb.at[pl.ds(0, ce)]],
                                  xlrows[b], seml[b]).wait()
            pltpu.make_async_copy(xr_hbm.at[dstb.at[pl.ds(0, ce)]],
                                  xrrows[b], semr[b]).wait()

        def compute(r, b):
            xlr = xlrows[b]
            xrr = xrrows[b]

            @pl.loop(0, ce, unroll=4)
            def _(e):
                accs = [jnp.zeros((LNS,), jnp.float32) for _ in range(4)]
                for j2 in range(nj):
                    sl = pl.ds(j2 * LNS, LNS)
                    a = plsc.bitcast(xlr[e, sl], jnp.bfloat16)
                    b2 = plsc.bitcast(xrr[e, sl], jnp.bfloat16)
                    z = a + b2
                    z = jnp.maximum(z, z * jnp.bfloat16(0.2))
                    prod = z * atts[j2]
                    pa, pb = plsc.unpack(prod,
                                         format=plsc.PackFormat.INTERLEAVED)
                    hh = j2 // (nj // 2)
                    accs[hh] = accs[hh] + pa
                    accs[2 + hh] = accs[2 + hh] + pb
                for h in range(4):
                    accb[pl.ds((e * 4 + h) * LNS, LNS)] = accs[h]

            for g in range(ce * 4 // LNS):
                tot = jnp.zeros((LNS,), jnp.float32)
                rbase = (g * LNS + iota) * LNS
                for cc in range(LNS):
                    tot = tot + plsc.load_gather(accb, [rbase + cc])
                pexpb[pl.ds(r * (ce * 4) + g * LNS, LNS)] = jnp.exp(tot)

            @pl.loop(0, ce, unroll=4)
            def _(e):
                dv = plsc.load_gather(dstsb,
                                      [jnp.zeros((LNS,), jnp.int32) + r * ce + e])
                didx = dv * 4 + h4
                old = plsc.load_gather(dloc, [didx])
                p16 = plsc.load_gather(pexpb, [(r * ce + e) * 4 + h4])
                plsc.store_scatter(dloc, [didx], old + p16, mask=m4)

        @pl.loop(0, nblk)
        def _(blk):
            eoff = base + blk * ib
            pltpu.sync_copy(src_hbm.at[pl.ds(eoff, ib)], srcb)
            pltpu.sync_copy(dstg_hbm.at[pl.ds(eoff, ib)], dstb)
            pltpu.sync_copy(dsts_hbm.at[pl.ds(eoff, ib)], dstsb)
            issue(0, 0)
            issue(1, 1)

            @pl.loop(0, blkc // nbuf)
            def _(kk):
                for b in range(nbuf):
                    r = kk * nbuf + b
                    wait_rows(b)

                    @pl.when(r + 2 < blkc)
                    def _():
                        issue(r + 2, (b + 2) % nbuf)
                    compute(r, b)

            pltpu.sync_copy(pexpb, pexp_hbm.at[pl.ds(eoff * 4, ib * 4)])

        pltpu.sync_copy(dloc, dpart_hbm.at[wid])

    return k(xlb, xrb, src, dstg, dsts, att_perm)


def _dreduce(dpart):
    nw, dl = dpart.shape
    pw = dl // NW

    @functools.partial(
        pl.kernel,
        compiler_params=pltpu.CompilerParams(needs_layout_passes=False),
        out_type=jax.ShapeDtypeStruct((dl,), jnp.float32),
        mesh=_MESH,
        scratch_types=[
            pltpu.VMEM((pw,), jnp.float32),
            pltpu.VMEM((pw,), jnp.float32),
        ],
    )
    def k(dp_hbm, dfull_hbm, acc, tmp):
        c = lax.axis_index("c")
        s = lax.axis_index("s")
        wid = s * NC + c
        sl = pl.ds(wid * pw, pw)
        pltpu.sync_copy(dp_hbm.at[0, sl], acc)
        for r in range(1, nw):
            pltpu.sync_copy(dp_hbm.at[r, sl], tmp)
            for t in range(pw // LNS):
                ssl = pl.ds(t * LNS, LNS)
                acc[ssl] = acc[ssl] + tmp[ssl]
        pltpu.sync_copy(acc, dfull_hbm.at[sl])

    return k(dpart)


def _aggregate(xl4, src, dsts, pexpf, z128, e_pad, npad, kc, d):
    ecw = e_pad // NS
    nch = ecw // kc
    rp = npad // NS
    blkc = 18
    nblk = nch // blkc
    ib = blkc * kc
    nbuf = 4

    @functools.partial(
        pl.kernel,
        compiler_params=pltpu.CompilerParams(needs_layout_passes=False),
        out_type=[jax.ShapeDtypeStruct((npad, d), jnp.float32)] * 4,
        mesh=_MESH,
        scratch_types=[
            pltpu.VMEM((ib,), jnp.int32),
            pltpu.VMEM((ib,), jnp.int32),
            [pltpu.VMEM((kc,), jnp.int32) for _ in range(nbuf)],
            pltpu.VMEM((ib * 4,), jnp.float32),
            pltpu.VMEM((kc,), jnp.float32),
            [pltpu.VMEM((kc, d), jnp.float32) for _ in range(nbuf)],
            pltpu.VMEM_SHARED((npad, d), jnp.float32),
            [pltpu.SemaphoreType.DMA for _ in range(nbuf)],
            [pltpu.SemaphoreType.DMA for _ in range(nbuf)],
        ],
    )
    def k(xl4_hbm, src_hbm, dst_hbm, pexp_hbm, z_hbm,
          o0, o1, o2, o3, gidxb, dstb, dstv, pvb, alph, xlr, osh,
          gsem, ssem):
        c = lax.axis_index("c")
        s = lax.axis_index("s")
        iota = _iota16()
        rsl = pl.ds(s * rp, rp)
        for hp in range(2):
            head = c * 2 + hp
            pltpu.sync_copy(z_hbm.at[pl.ds(0, rp)], osh.at[rsl])
            plsc.subcore_barrier()

            def issue(r, b):
                pltpu.async_copy(xl4_hbm.at[gidxb.at[pl.ds(r * kc, kc)]],
                                 xlr[b], gsem[b])

            def wait_rows(b):
                pltpu.make_async_copy(xl4_hbm.at[gidxb.at[pl.ds(0, kc)]],
                                      xlr[b], gsem[b]).wait()

            def wait_scatter(b):
                pltpu.make_async_copy(xlr[b], osh.at[dstv[b]],
                                      ssem[b]).wait()

            def compute_and_scatter(r, b):
                for g in range(kc // LNS):
                    gsl = pl.ds(g * LNS, LNS)
                    e16 = (r * kc + iota + g * LNS) * 4 + head
                    alph[gsl] = plsc.load_gather(pvb, [e16])
                    dstv[b][gsl] = dstb[pl.ds(r * kc + g * LNS, LNS)]
                xb = xlr[b]

                @pl.loop(0, kc, unroll=4)
                def _(e):
                    av = plsc.load_gather(alph,
                                          [jnp.zeros((LNS,), jnp.int32) + e])
                    for j2 in range(d // LNS):
                        jsl = pl.ds(j2 * LNS, LNS)
                        xb[e, jsl] = xb[e, jsl] * av

                pltpu.async_copy(xb, osh.at[dstv[b]], ssem[b], add=True)

            @pl.loop(0, nblk)
            def _(blk):
                eoff = s * ecw + blk * ib
                pltpu.sync_copy(src_hbm.at[pl.ds(eoff, ib)], gidxb)
                pltpu.sync_copy(dst_hbm.at[pl.ds(eoff, ib)], dstb)
                pltpu.sync_copy(pexp_hbm.at[pl.ds(eoff * 4, ib * 4)], pvb)
                for g in range(ib // LNS):
                    gsl = pl.ds(g * LNS, LNS)
                    gidxb[gsl] = gidxb[gsl] * 4 + head
                issue(0, 0)
                issue(1, 1)

                @pl.loop(0, blkc // nbuf)
                def _(kk):
                    for b in range(nbuf):
                        r = kk * nbuf + b
                        wait_rows(b)
                        nb = (b + 2) % nbuf

                        @pl.when(jnp.logical_and(r >= 2, r + 2 < blkc))
                        def _():
                            wait_scatter(nb)

                        @pl.when(r + 2 < blkc)
                        def _():
                            issue(r + 2, nb)
                        compute_and_scatter(r, b)

                for r in (blkc - 2, blkc - 1):
                    b = r % nbuf
                    wait_rows(b)
                    compute_and_scatter(r, b)

                for r in range(blkc - 4, blkc):
                    wait_scatter(r % nbuf)

            plsc.subcore_barrier()
            if hp == 0:
                @pl.when(c == 0)
                def _():
                    pltpu.sync_copy(osh.at[rsl], o0.at[rsl])

                @pl.when(c == 1)
                def _():
                    pltpu.sync_copy(osh.at[rsl], o2.at[rsl])
            else:
                @pl.when(c == 0)
                def _():
                    pltpu.sync_copy(osh.at[rsl], o1.at[rsl])

                @pl.when(c == 1)
                def _():
                    pltpu.sync_copy(osh.at[rsl], o3.at[rsl])
            plsc.subcore_barrier()

    return k(xl4, src, dsts, pexpf, z128)


def _post(oheads, dpart, inp, W_lin, b_lin, gat2d, ln_w, ln_b):
    n, d = inp.shape
    hd = W_lin.shape[1]
    BR = 1000

    def body(o0_ref, o1_ref, o2_ref, o3_ref, dp_ref, inp_ref,
             wl_ref, bl_ref, gb_ref, lw_ref, lb_ref, out_ref):
        dsum = dp_ref[...]
        acc = None
        wl = wl_ref[...]
        for h, oref in enumerate([o0_ref, o1_ref, o2_ref, o3_ref]):
            oh = oref[...] / dsum[:, h:h + 1] + gb_ref[h, :][None, :]
            t = lax.dot_general(oh, wl[:, h * d:(h + 1) * d], _DN,
                                preferred_element_type=jnp.float32)
            acc = t if acc is None else acc + t
        y = jnp.maximum(acc + bl_ref[...], 0.0)
        r = y + inp_ref[...]
        u = jnp.mean(r, axis=1, keepdims=True)
        var = jnp.mean((r - u) ** 2, axis=1, keepdims=True)
        out_ref[...] = lw_ref[...] * ((r - u) / jnp.sqrt(var + 1e-12)) + lb_ref[...]

    row_spec = pl.BlockSpec((BR, d), lambda i: (i, 0))
    full = lambda shape: pl.BlockSpec(shape, lambda i: (0,) * len(shape))
    return pl.pallas_call(
        body,
        grid=(n // BR,),
        in_specs=[row_spec, row_spec, row_spec, row_spec,
                  pl.BlockSpec((BR, 4), lambda i: (i, 0)),
                  row_spec,
                  full((d, hd)), full((1, d)), full((4, d)),
                  full((1, d)), full((1, d))],
        out_specs=row_spec,
        out_shape=jax.ShapeDtypeStruct((n, d), jnp.float32),
    )(*oheads, dpart, inp, W_lin, b_lin.reshape(1, d), gat2d,
      ln_w.reshape(1, d), ln_b.reshape(1, d))


def kernel(x, edge_index, W_AE, b_AE, Wl, bl, Wr, br, att, gat_bias,
           W_lin, b_lin, ln_w, ln_b):
    n, d = x.shape
    e = edge_index.shape[1]
    nheads = att.shape[0]

    inp, xl, xlb, xrb = _proj(x, W_AE, b_AE, Wl, bl, Wr, br)

    src0 = edge_index[0].astype(jnp.int32)
    dst0 = edge_index[1].astype(jnp.int32)
    loop = jnp.arange(n, dtype=jnp.int32)
    etot = e + n
    ce, kc = 32, 64
    ew = -(-etot // (NW * 128)) * 128
    e_pad = ew * NW
    pad = e_pad - etot
    src = jnp.concatenate([src0, loop, jnp.zeros((pad,), jnp.int32)])
    dstg = jnp.concatenate([dst0, loop, jnp.zeros((pad,), jnp.int32)])
    dsts = jnp.concatenate([dst0, loop, jnp.full((pad,), n, jnp.int32)])
    npad = -(-(n + 1) // 256) * 256

    hd2 = nheads * d // 2
    att_bf = att.reshape(-1).astype(jnp.bfloat16)
    attp = lax.bitcast_convert_type(
        jnp.stack([att_bf[:hd2], att_bf[hd2:]], axis=-1), jnp.int32)

    pexpf, dpart = _edge_logits(xlb, xrb, src, dstg, dsts, attp,
                                e_pad, ew, ce, npad)

    dfull = _dreduce(dpart)

    z128 = jnp.zeros((npad, d), jnp.float32)
    o = _aggregate(xl.reshape(n * nheads, d), src, dsts, pexpf, z128,
                   e_pad, npad, kc, d)

    return _post(list(o), dfull.reshape(npad, 4),
                 inp, W_lin, b_lin, gat_bias.reshape(nheads, d), ln_w, ln_b)


# speedup vs baseline: 1.0126x; 1.0126x over previous
import functools

import jax
import jax.numpy as jnp
from jax import lax
from jax.experimental import pallas as pl
from jax.experimental.pallas import tpu as pltpu
from jax.experimental.pallas import tpu_sc as plsc

NC = 2
NS = 16
NW = NC * NS
LNS = 16

_MESH = plsc.VectorSubcoreMesh(core_axis_name="c", subcore_axis_name="s")
_DN = (((1,), (1,)), ((), ()))


def _iota16():
    return lax.broadcasted_iota(jnp.int32, (LNS,), 0)


def _proj(x, W_AE, b_AE, Wl, bl, Wr, br):
    n, d = x.shape
    hd = Wl.shape[0]
    BR = 1000

    def pack_half_rows(v):
        b16 = v.astype(jnp.bfloat16)
        lo = lax.bitcast_convert_type(b16[:, :hd // 2],
                                      jnp.uint16).astype(jnp.uint32)
        hi = lax.bitcast_convert_type(b16[:, hd // 2:],
                                      jnp.uint16).astype(jnp.uint32)
        return lax.bitcast_convert_type(lo | (hi << 16), jnp.int32)

    def body(x_ref, wae_ref, bae_ref, wl_ref, bl_ref, wr_ref, br_ref,
             inp_ref, xl_ref, xlb_ref, xrb_ref):
        xv = x_ref[...]
        inp = lax.dot_general(xv, wae_ref[...], _DN,
                              preferred_element_type=jnp.float32) + bae_ref[...]
        inp_ref[...] = inp
        xlv = lax.dot_general(inp, wl_ref[...], _DN,
                              preferred_element_type=jnp.float32) + bl_ref[...]
        xl_ref[...] = xlv
        xlb_ref[...] = pack_half_rows(xlv)
        xrv = lax.dot_general(inp, wr_ref[...], _DN,
                              preferred_element_type=jnp.float32) + br_ref[...]
        xrb_ref[...] = pack_half_rows(xrv)

    return pl.pallas_call(
        body,
        grid=(n // BR,),
        in_specs=[
            pl.BlockSpec((BR, d), lambda i: (i, 0)),
            pl.BlockSpec((d, d), lambda i: (0, 0)),
            pl.BlockSpec((1, d), lambda i: (0, 0)),
            pl.BlockSpec((hd, d), lambda i: (0, 0)),
            pl.BlockSpec((1, hd), lambda i: (0, 0)),
            pl.BlockSpec((hd, d), lambda i: (0, 0)),
            pl.BlockSpec((1, hd), lambda i: (0, 0)),
        ],
        out_specs=[
            pl.BlockSpec((BR, d), lambda i: (i, 0)),
            pl.BlockSpec((BR, hd), lambda i: (i, 0)),
            pl.BlockSpec((BR, hd // 2), lambda i: (i, 0)),
            pl.BlockSpec((BR, hd // 2), lambda i: (i, 0)),
        ],
        out_shape=[
            jax.ShapeDtypeStruct((n, d), jnp.float32),
            jax.ShapeDtypeStruct((n, hd), jnp.float32),
            jax.ShapeDtypeStruct((n, hd // 2), jnp.int32),
            jax.ShapeDtypeStruct((n, hd // 2), jnp.int32),
        ],
    )(x, W_AE, b_AE.reshape(1, d), Wl, bl.reshape(1, hd), Wr, br.reshape(1, hd))


def _edge_logits(xlb, xrb, src, dstg, dsts, att_perm, e_pad, ew, ce, npad):
    hdw = xlb.shape[1]
    hd = hdw * 2
    nch = ew // ce
    nj = hd // (2 * LNS)
    dl = npad * 4

    blkc = 24
    nblk = nch // blkc
    ib = blkc * ce
    nbuf = 3

    @functools.partial(
        pl.kernel,
        compiler_params=pltpu.CompilerParams(needs_layout_passes=False),
        out_type=[
            jax.ShapeDtypeStruct((e_pad * 4,), jnp.float32),
            jax.ShapeDtypeStruct((NW, dl), jnp.float32),
        ],
        mesh=_MESH,
        scratch_types=[
            pltpu.VMEM((hdw,), jnp.int32),
            pltpu.VMEM((ib,), jnp.int32),
            pltpu.VMEM((ib,), jnp.int32),
            pltpu.VMEM((ib,), jnp.int32),
            [pltpu.VMEM((ce, hdw), jnp.int32) for _ in range(nbuf)],
            [pltpu.VMEM((ce, hdw), jnp.int32) for _ in range(nbuf)],
            pltpu.VMEM((ce * 4 * LNS,), jnp.float32),
            pltpu.VMEM((ib * 4,), jnp.float32),
            pltpu.VMEM((dl,), jnp.float32),
            [pltpu.SemaphoreType.DMA for _ in range(nbuf)],
            [pltpu.SemaphoreType.DMA for _ in range(nbuf)],
        ],
    )
    def k(xl_hbm, xr_hbm, src_hbm, dstg_hbm, dsts_hbm, att_hbm,
          pexp_hbm, dpart_hbm,
          attv, srcb, dstb, dstsb, xlrows, xrrows, accb, pexpb, dloc,
          seml, semr):
        c = lax.axis_index("c")
        s = lax.axis_index("s")
        wid = s * NC + c
        base = wid * ew
        iota = _iota16()
        h4 = jnp.minimum(iota, 3)
        m4 = iota < 4

        zv = jnp.zeros((LNS,), jnp.float32)

        @pl.loop(0, dl // LNS)
        def _(t):
            dloc[pl.ds(t * LNS, LNS)] = zv
        pltpu.sync_copy(att_hbm, attv)
        atts = [plsc.bitcast(attv[pl.ds(j * LNS, LNS)], jnp.bfloat16)
                for j in range(nj)]

        def issue(r, b):
            pltpu.async_copy(xl_hbm.at[srcb.at[pl.ds(r * ce, ce)]],
                             xlrows[b], seml[b])
            pltpu.async_copy(xr_hbm.at[dstb.at[pl.ds(r * ce, ce)]],
                             xrrows[b], semr[b])

        def wait_rows(b):
            pltpu.make_async_copy(xl_hbm.at[srcb.at[pl.ds(0, ce)]],
                                  xlrows[b], seml[b]).wait()
            pltpu.make_async_copy(xr_hbm.at[dstb.at[pl.ds(0, ce)]],
                                  xrrows[b], semr[b]).wait()

        def compute(r, b):
            xlr = xlrows[b]
            xrr = xrrows[b]

            @pl.loop(0, ce, unroll=4)
            def _(e):
                accs = [jnp.zeros((LNS,), jnp.float32) for _ in range(4)]
                for j2 in range(nj):
                    sl = pl.ds(j2 * LNS, LNS)
                    a = plsc.bitcast(xlr[e, sl], jnp.bfloat16)
                    b2 = plsc.bitcast(xrr[e, sl], jnp.bfloat16)
                    z = a + b2
                    z = jnp.maximum(z, z * jnp.bfloat16(0.2))
                    prod = z * atts[j2]
                    pa, pb = plsc.unpack(prod,
                                         format=plsc.PackFormat.INTERLEAVED)
                    hh = j2 // (nj // 2)
                    accs[hh] = accs[hh] + pa
                    accs[2 + hh] = accs[2 + hh] + pb
                for h in range(4):
                    accb[pl.ds((e * 4 + h) * LNS, LNS)] = accs[h]

            for g in range(ce * 4 // LNS):
                tot = jnp.zeros((LNS,), jnp.float32)
                rbase = (g * LNS + iota) * LNS
                for cc in range(LNS):
                    tot = tot + plsc.load_gather(accb, [rbase + cc])
                pexpb[pl.ds(r * (ce * 4) + g * LNS, LNS)] = jnp.exp(tot)

            @pl.loop(0, ce, unroll=4)
            def _(e):
                dv = plsc.load_gather(dstsb,
                                      [jnp.zeros((LNS,), jnp.int32) + r * ce + e])
                didx = dv * 4 + h4
                old = plsc.load_gather(dloc, [didx])
                p16 = plsc.load_gather(pexpb, [(r * ce + e) * 4 + h4])
                plsc.store_scatter(dloc, [didx], old + p16, mask=m4)

        @pl.loop(0, nblk)
        def _(blk):
            eoff = base + blk * ib
            pltpu.sync_copy(src_hbm.at[pl.ds(eoff, ib)], srcb)
            pltpu.sync_copy(dstg_hbm.at[pl.ds(eoff, ib)], dstb)
            pltpu.sync_copy(dsts_hbm.at[pl.ds(eoff, ib)], dstsb)
            issue(0, 0)
            issue(1, 1)

            @pl.loop(0, blkc // nbuf)
            def _(kk):
                for b in range(nbuf):
                    r = kk * nbuf + b
                    wait_rows(b)

                    @pl.when(r + 2 < blkc)
                    def _():
                        issue(r + 2, (b + 2) % nbuf)
                    compute(r, b)

            pltpu.sync_copy(pexpb, pexp_hbm.at[pl.ds(eoff * 4, ib * 4)])

        pltpu.sync_copy(dloc, dpart_hbm.at[wid])

    return k(xlb, xrb, src, dstg, dsts, att_perm)


def _dreduce(dpart):
    nw, dl = dpart.shape
    pw = dl // NW

    @functools.partial(
        pl.kernel,
        compiler_params=pltpu.CompilerParams(needs_layout_passes=False),
        out_type=jax.ShapeDtypeStruct((dl,), jnp.float32),
        mesh=_MESH,
        scratch_types=[
            pltpu.VMEM((pw,), jnp.float32),
            pltpu.VMEM((pw,), jnp.float32),
        ],
    )
    def k(dp_hbm, dfull_hbm, acc, tmp):
        c = lax.axis_index("c")
        s = lax.axis_index("s")
        wid = s * NC + c
        sl = pl.ds(wid * pw, pw)
        pltpu.sync_copy(dp_hbm.at[0, sl], acc)
        for r in range(1, nw):
            pltpu.sync_copy(dp_hbm.at[r, sl], tmp)
            for t in range(pw // LNS):
                ssl = pl.ds(t * LNS, LNS)
                acc[ssl] = acc[ssl] + tmp[ssl]
        pltpu.sync_copy(acc, dfull_hbm.at[sl])

    return k(dpart)


def _aggregate(xl4, src, dsts, pexpf, z128, e_pad, npad, kc, d):
    ecw = e_pad // NS
    nch = ecw // kc
    rp = npad // NS
    blkc = 18
    nblk = nch // blkc
    ib = blkc * kc
    nbuf = 4

    @functools.partial(
        pl.kernel,
        compiler_params=pltpu.CompilerParams(needs_layout_passes=False),
        out_type=[jax.ShapeDtypeStruct((npad, d), jnp.float32)] * 4,
        mesh=_MESH,
        scratch_types=[
            pltpu.VMEM((ib,), jnp.int32),
            pltpu.VMEM((ib,), jnp.int32),
            [pltpu.VMEM((kc,), jnp.int32) for _ in range(nbuf)],
            pltpu.VMEM((ib * 4,), jnp.float32),
            pltpu.VMEM((kc,), jnp.float32),
            [pltpu.VMEM((kc, d), jnp.float32) for _ in range(nbuf)],
            pltpu.VMEM_SHARED((npad, d), jnp.float32),
            [pltpu.SemaphoreType.DMA for _ in range(nbuf)],
            [pltpu.SemaphoreType.DMA for _ in range(nbuf)],
        ],
    )
    def k(xl4_hbm, src_hbm, dst_hbm, pexp_hbm, z_hbm,
          o0, o1, o2, o3, gidxb, dstb, dstv, pvb, alph, xlr, osh,
          gsem, ssem):
        c = lax.axis_index("c")
        s = lax.axis_index("s")
        iota = _iota16()
        rsl = pl.ds(s * rp, rp)
        for hp in range(2):
            head = c * 2 + hp
            pltpu.sync_copy(z_hbm.at[pl.ds(0, rp)], osh.at[rsl])
            plsc.subcore_barrier()

            def issue(r, b):
                pltpu.async_copy(xl4_hbm.at[gidxb.at[pl.ds(r * kc, kc)]],
                                 xlr[b], gsem[b])

            def wait_rows(b):
                pltpu.make_async_copy(xl4_hbm.at[gidxb.at[pl.ds(0, kc)]],
                                      xlr[b], gsem[b]).wait()

            def wait_scatter(b):
                pltpu.make_async_copy(xlr[b], osh.at[dstv[b]],
                                      ssem[b]).wait()

            def compute_and_scatter(r, b):
                for g in range(kc // LNS):
                    gsl = pl.ds(g * LNS, LNS)
                    e16 = (r * kc + iota + g * LNS) * 4 + head
                    alph[gsl] = plsc.load_gather(pvb, [e16])
                    dstv[b][gsl] = dstb[pl.ds(r * kc + g * LNS, LNS)]
                xb = xlr[b]

                @pl.loop(0, kc, unroll=4)
                def _(e):
                    av = plsc.load_gather(alph,
                                          [jnp.zeros((LNS,), jnp.int32) + e])
                    for j2 in range(d // LNS):
                        jsl = pl.ds(j2 * LNS, LNS)
                        xb[e, jsl] = xb[e, jsl] * av

                pltpu.async_copy(xb, osh.at[dstv[b]], ssem[b], add=True)

            @pl.loop(0, nblk)
            def _(blk):
                eoff = s * ecw + blk * ib
                pltpu.sync_copy(src_hbm.at[pl.ds(eoff, ib)], gidxb)
                pltpu.sync_copy(dst_hbm.at[pl.ds(eoff, ib)], dstb)
                pltpu.sync_copy(pexp_hbm.at[pl.ds(eoff * 4, ib * 4)], pvb)
                for g in range(ib // LNS):
                    gsl = pl.ds(g * LNS, LNS)
                    gidxb[gsl] = gidxb[gsl] * 4 + head
                issue(0, 0)
                issue(1, 1)

                @pl.loop(0, blkc // nbuf)
                def _(kk):
                    for b in range(nbuf):
                        r = kk * nbuf + b
                        wait_rows(b)
                        nb = (b + 2) % nbuf

                        @pl.when(jnp.logical_and(r >= 2, r + 2 < blkc))
                        def _():
                            wait_scatter(nb)

                        @pl.when(r + 2 < blkc)
                        def _():
                            issue(r + 2, nb)
                        compute_and_scatter(r, b)

                for r in (blkc - 2, blkc - 1):
                    b = r % nbuf
                    wait_rows(b)
                    compute_and_scatter(r, b)

                for r in range(blkc - 4, blkc):
                    wait_scatter(r % nbuf)

            plsc.subcore_barrier()
            if hp == 0:
                @pl.when(c == 0)
                def _():
                    pltpu.sync_copy(osh.at[rsl], o0.at[rsl])

                @pl.when(c == 1)
                def _():
                    pltpu.sync_copy(osh.at[rsl], o2.at[rsl])
            else:
                @pl.when(c == 0)
                def _():
                    pltpu.sync_copy(osh.at[rsl], o1.at[rsl])

                @pl.when(c == 1)
                def _():
                    pltpu.sync_copy(osh.at[rsl], o3.at[rsl])
            plsc.subcore_barrier()

    return k(xl4, src, dsts, pexpf, z128)


def _post(oheads, dpart, inp, W_lin, b_lin, gat2d, ln_w, ln_b):
    n, d = inp.shape
    hd = W_lin.shape[1]
    BR = 1000

    def body(o0_ref, o1_ref, o2_ref, o3_ref, dp_ref, inp_ref,
             wl_ref, bl_ref, gb_ref, lw_ref, lb_ref, out_ref):
        dsum = dp_ref[...]
        acc = None
        wl = wl_ref[...]
        for h, oref in enumerate([o0_ref, o1_ref, o2_ref, o3_ref]):
            oh = oref[...] / dsum[:, h:h + 1] + gb_ref[h, :][None, :]
            t = lax.dot_general(oh, wl[:, h * d:(h + 1) * d], _DN,
                                preferred_element_type=jnp.float32)
            acc = t if acc is None else acc + t
        y = jnp.maximum(acc + bl_ref[...], 0.0)
        r = y + inp_ref[...]
        u = jnp.mean(r, axis=1, keepdims=True)
        var = jnp.mean((r - u) ** 2, axis=1, keepdims=True)
        out_ref[...] = lw_ref[...] * ((r - u) / jnp.sqrt(var + 1e-12)) + lb_ref[...]

    row_spec = pl.BlockSpec((BR, d), lambda i: (i, 0))
    full = lambda shape: pl.BlockSpec(shape, lambda i: (0,) * len(shape))
    return pl.pallas_call(
        body,
        grid=(n // BR,),
        in_specs=[row_spec, row_spec, row_spec, row_spec,
                  pl.BlockSpec((BR, 4), lambda i: (i, 0)),
                  row_spec,
                  full((d, hd)), full((1, d)), full((4, d)),
                  full((1, d)), full((1, d))],
        out_specs=row_spec,
        out_shape=jax.ShapeDtypeStruct((n, d), jnp.float32),
    )(*oheads, dpart, inp, W_lin, b_lin.reshape(1, d), gat2d,
      ln_w.reshape(1, d), ln_b.reshape(1, d))


def kernel(x, edge_index, W_AE, b_AE, Wl, bl, Wr, br, att, gat_bias,
           W_lin, b_lin, ln_w, ln_b):
    n, d = x.shape
    e = edge_index.shape[1]
    nheads = att.shape[0]

    inp, xl, xlb, xrb = _proj(x, W_AE, b_AE, Wl, bl, Wr, br)

    src0 = edge_index[0].astype(jnp.int32)
    dst0 = edge_index[1].astype(jnp.int32)
    loop = jnp.arange(n, dtype=jnp.int32)
    etot = e + n
    ce, kc = 48, 64
    ew = -(-etot // (NW * 128)) * 128
    e_pad = ew * NW
    pad = e_pad - etot
    src = jnp.concatenate([src0, loop, jnp.zeros((pad,), jnp.int32)])
    dstg = jnp.concatenate([dst0, loop, jnp.zeros((pad,), jnp.int32)])
    dsts = jnp.concatenate([dst0, loop, jnp.full((pad,), n, jnp.int32)])
    npad = -(-(n + 1) // 256) * 256

    hd2 = nheads * d // 2
    att_bf = att.reshape(-1).astype(jnp.bfloat16)
    attp = lax.bitcast_convert_type(
        jnp.stack([att_bf[:hd2], att_bf[hd2:]], axis=-1), jnp.int32)

    pexpf, dpart = _edge_logits(xlb, xrb, src, dstg, dsts, attp,
                                e_pad, ew, ce, npad)

    dfull = _dreduce(dpart)

    z128 = jnp.zeros((npad, d), jnp.float32)
    o = _aggregate(xl.reshape(n * nheads, d), src, dsts, pexpf, z128,
                   e_pad, npad, kc, d)

    return _post(list(o), dfull.reshape(npad, 4),
                 inp, W_lin, b_lin, gat_bias.reshape(nheads, d), ln_w, ln_b)


# speedup vs baseline: 1.0167x; 1.0041x over previous
import functools

import jax
import jax.numpy as jnp
from jax import lax
from jax.experimental import pallas as pl
from jax.experimental.pallas import tpu as pltpu
from jax.experimental.pallas import tpu_sc as plsc

NC = 2
NS = 16
NW = NC * NS
LNS = 16

_MESH = plsc.VectorSubcoreMesh(core_axis_name="c", subcore_axis_name="s")
_DN = (((1,), (1,)), ((), ()))


def _iota16():
    return lax.broadcasted_iota(jnp.int32, (LNS,), 0)


def _proj(x, W_AE, b_AE, Wl, bl, Wr, br):
    n, d = x.shape
    hd = Wl.shape[0]
    BR = 1000

    def pack_half_rows(v):
        b16 = v.astype(jnp.bfloat16)
        lo = lax.bitcast_convert_type(b16[:, :hd // 2],
                                      jnp.uint16).astype(jnp.uint32)
        hi = lax.bitcast_convert_type(b16[:, hd // 2:],
                                      jnp.uint16).astype(jnp.uint32)
        return lax.bitcast_convert_type(lo | (hi << 16), jnp.int32)

    def body(x_ref, wae_ref, bae_ref, wl_ref, bl_ref, wr_ref, br_ref,
             inp_ref, xl_ref, xlb_ref, xrb_ref):
        xv = x_ref[...]
        inp = lax.dot_general(xv, wae_ref[...], _DN,
                              preferred_element_type=jnp.float32) + bae_ref[...]
        inp_ref[...] = inp
        xlv = lax.dot_general(inp, wl_ref[...], _DN,
                              preferred_element_type=jnp.float32) + bl_ref[...]
        xl_ref[...] = xlv
        xlb_ref[...] = pack_half_rows(xlv)
        xrv = lax.dot_general(inp, wr_ref[...], _DN,
                              preferred_element_type=jnp.float32) + br_ref[...]
        xrb_ref[...] = pack_half_rows(xrv)

    return pl.pallas_call(
        body,
        grid=(n // BR,),
        in_specs=[
            pl.BlockSpec((BR, d), lambda i: (i, 0)),
            pl.BlockSpec((d, d), lambda i: (0, 0)),
            pl.BlockSpec((1, d), lambda i: (0, 0)),
            pl.BlockSpec((hd, d), lambda i: (0, 0)),
            pl.BlockSpec((1, hd), lambda i: (0, 0)),
            pl.BlockSpec((hd, d), lambda i: (0, 0)),
            pl.BlockSpec((1, hd), lambda i: (0, 0)),
        ],
        out_specs=[
            pl.BlockSpec((BR, d), lambda i: (i, 0)),
            pl.BlockSpec((BR, hd), lambda i: (i, 0)),
            pl.BlockSpec((BR, hd // 2), lambda i: (i, 0)),
            pl.BlockSpec((BR, hd // 2), lambda i: (i, 0)),
        ],
        out_shape=[
            jax.ShapeDtypeStruct((n, d), jnp.float32),
            jax.ShapeDtypeStruct((n, hd), jnp.float32),
            jax.ShapeDtypeStruct((n, hd // 2), jnp.int32),
            jax.ShapeDtypeStruct((n, hd // 2), jnp.int32),
        ],
    )(x, W_AE, b_AE.reshape(1, d), Wl, bl.reshape(1, hd), Wr, br.reshape(1, hd))


def _edge_logits(xlb, xrb, src, dstg, dsts, att_perm, e_pad, ew, ce, npad):
    hdw = xlb.shape[1]
    hd = hdw * 2
    nch = ew // ce
    nj = hd // (2 * LNS)
    dl = npad * 4

    blkc = 18
    nblk = nch // blkc
    ib = blkc * ce
    nbuf = 2

    @functools.partial(
        pl.kernel,
        compiler_params=pltpu.CompilerParams(needs_layout_passes=False),
        out_type=[
            jax.ShapeDtypeStruct((e_pad * 4,), jnp.float32),
            jax.ShapeDtypeStruct((NW, dl), jnp.float32),
        ],
        mesh=_MESH,
        scratch_types=[
            pltpu.VMEM((hdw,), jnp.int32),
            pltpu.VMEM((ib,), jnp.int32),
            pltpu.VMEM((ib,), jnp.int32),
            pltpu.VMEM((ib,), jnp.int32),
            [pltpu.VMEM((ce, hdw), jnp.int32) for _ in range(nbuf)],
            [pltpu.VMEM((ce, hdw), jnp.int32) for _ in range(nbuf)],
            pltpu.VMEM((ce * 4 * LNS,), jnp.float32),
            pltpu.VMEM((ib * 4,), jnp.float32),
            pltpu.VMEM((dl,), jnp.float32),
            [pltpu.SemaphoreType.DMA for _ in range(nbuf)],
            [pltpu.SemaphoreType.DMA for _ in range(nbuf)],
        ],
    )
    def k(xl_hbm, xr_hbm, src_hbm, dstg_hbm, dsts_hbm, att_hbm,
          pexp_hbm, dpart_hbm,
          attv, srcb, dstb, dstsb, xlrows, xrrows, accb, pexpb, dloc,
          seml, semr):
        c = lax.axis_index("c")
        s = lax.axis_index("s")
        wid = s * NC + c
        base = wid * ew
        iota = _iota16()
        h4 = jnp.minimum(iota, 3)
        m4 = iota < 4

        zv = jnp.zeros((LNS,), jnp.float32)

        @pl.loop(0, dl // LNS)
        def _(t):
            dloc[pl.ds(t * LNS, LNS)] = zv
        pltpu.sync_copy(att_hbm, attv)
        atts = [plsc.bitcast(attv[pl.ds(j * LNS, LNS)], jnp.bfloat16)
                for j in range(nj)]

        def issue(r, b):
            pltpu.async_copy(xl_hbm.at[srcb.at[pl.ds(r * ce, ce)]],
                             xlrows[b], seml[b])
            pltpu.async_copy(xr_hbm.at[dstb.at[pl.ds(r * ce, ce)]],
                             xrrows[b], semr[b])

        def wait_rows(b):
            pltpu.make_async_copy(xl_hbm.at[srcb.at[pl.ds(0, ce)]],
                                  xlrows[b], seml[b]).wait()
            pltpu.make_async_copy(xr_hbm.at[dstb.at[pl.ds(0, ce)]],
                                  xrrows[b], semr[b]).wait()

        def compute(r, b):
            xlr = xlrows[b]
            xrr = xrrows[b]

            @pl.loop(0, ce, unroll=4)
            def _(e):
                accs = [jnp.zeros((LNS,), jnp.float32) for _ in range(4)]
                for j2 in range(nj):
                    sl = pl.ds(j2 * LNS, LNS)
                    a = plsc.bitcast(xlr[e, sl], jnp.bfloat16)
                    b2 = plsc.bitcast(xrr[e, sl], jnp.bfloat16)
                    z = a + b2
                    z = jnp.maximum(z, z * jnp.bfloat16(0.2))
                    prod = z * atts[j2]
                    pa, pb = plsc.unpack(prod,
                                         format=plsc.PackFormat.INTERLEAVED)
                    hh = j2 // (nj // 2)
                    accs[hh] = accs[hh] + pa
                    accs[2 + hh] = accs[2 + hh] + pb
                for h in range(4):
                    accb[pl.ds((e * 4 + h) * LNS, LNS)] = accs[h]

            for g in range(ce * 4 // LNS):
                tot = jnp.zeros((LNS,), jnp.float32)
                rbase = (g * LNS + iota) * LNS
                for cc in range(LNS):
                    tot = tot + plsc.load_gather(accb, [rbase + cc])
                pexpb[pl.ds(r * (ce * 4) + g * LNS, LNS)] = jnp.exp(tot)

            @pl.loop(0, ce, unroll=4)
            def _(e):
                dv = plsc.load_gather(dstsb,
                                      [jnp.zeros((LNS,), jnp.int32) + r * ce + e])
                didx = dv * 4 + h4
                old = plsc.load_gather(dloc, [didx])
                p16 = plsc.load_gather(pexpb, [(r * ce + e) * 4 + h4])
                plsc.store_scatter(dloc, [didx], old + p16, mask=m4)

        @pl.loop(0, nblk)
        def _(blk):
            eoff = base + blk * ib
            pltpu.sync_copy(src_hbm.at[pl.ds(eoff, ib)], srcb)
            pltpu.sync_copy(dstg_hbm.at[pl.ds(eoff, ib)], dstb)
            pltpu.sync_copy(dsts_hbm.at[pl.ds(eoff, ib)], dstsb)
            issue(0, 0)

            @pl.loop(0, blkc // 2)
            def _(kk):
                for b in range(2):
                    r = kk * 2 + b
                    wait_rows(b)

                    @pl.when(r + 1 < blkc)
                    def _():
                        issue(r + 1, 1 - b)
                    compute(r, b)

            pltpu.sync_copy(pexpb, pexp_hbm.at[pl.ds(eoff * 4, ib * 4)])

        pltpu.sync_copy(dloc, dpart_hbm.at[wid])

    return k(xlb, xrb, src, dstg, dsts, att_perm)


def _dreduce(dpart):
    nw, dl = dpart.shape
    pw = dl // NW

    @functools.partial(
        pl.kernel,
        compiler_params=pltpu.CompilerParams(needs_layout_passes=False),
        out_type=jax.ShapeDtypeStruct((dl,), jnp.float32),
        mesh=_MESH,
        scratch_types=[
            pltpu.VMEM((pw,), jnp.float32),
            pltpu.VMEM((pw,), jnp.float32),
        ],
    )
    def k(dp_hbm, dfull_hbm, acc, tmp):
        c = lax.axis_index("c")
        s = lax.axis_index("s")
        wid = s * NC + c
        sl = pl.ds(wid * pw, pw)
        pltpu.sync_copy(dp_hbm.at[0, sl], acc)
        for r in range(1, nw):
            pltpu.sync_copy(dp_hbm.at[r, sl], tmp)
            for t in range(pw // LNS):
                ssl = pl.ds(t * LNS, LNS)
                acc[ssl] = acc[ssl] + tmp[ssl]
        pltpu.sync_copy(acc, dfull_hbm.at[sl])

    return k(dpart)


def _aggregate(xl4, src, dsts, pexpf, z128, e_pad, npad, kc, d):
    ecw = e_pad // NS
    nch = ecw // kc
    rp = npad // NS
    blkc = 18
    nblk = nch // blkc
    ib = blkc * kc
    nbuf = 4

    @functools.partial(
        pl.kernel,
        compiler_params=pltpu.CompilerParams(needs_layout_passes=False),
        out_type=[jax.ShapeDtypeStruct((npad, d), jnp.float32)] * 4,
        mesh=_MESH,
        scratch_types=[
            pltpu.VMEM((ib,), jnp.int32),
            pltpu.VMEM((ib,), jnp.int32),
            [pltpu.VMEM((kc,), jnp.int32) for _ in range(nbuf)],
            pltpu.VMEM((ib * 4,), jnp.float32),
            pltpu.VMEM((kc,), jnp.float32),
            [pltpu.VMEM((kc, d), jnp.float32) for _ in range(nbuf)],
            pltpu.VMEM_SHARED((npad, d), jnp.float32),
            [pltpu.SemaphoreType.DMA for _ in range(nbuf)],
            [pltpu.SemaphoreType.DMA for _ in range(nbuf)],
        ],
    )
    def k(xl4_hbm, src_hbm, dst_hbm, pexp_hbm, z_hbm,
          o0, o1, o2, o3, gidxb, dstb, dstv, pvb, alph, xlr, osh,
          gsem, ssem):
        c = lax.axis_index("c")
        s = lax.axis_index("s")
        iota = _iota16()
        rsl = pl.ds(s * rp, rp)
        for hp in range(2):
            head = c * 2 + hp
            pltpu.sync_copy(z_hbm.at[pl.ds(0, rp)], osh.at[rsl])
            plsc.subcore_barrier()

            def issue(r, b):
                pltpu.async_copy(xl4_hbm.at[gidxb.at[pl.ds(r * kc, kc)]],
                                 xlr[b], gsem[b])

            def wait_rows(b):
                pltpu.make_async_copy(xl4_hbm.at[gidxb.at[pl.ds(0, kc)]],
                                      xlr[b], gsem[b]).wait()

            def wait_scatter(b):
                pltpu.make_async_copy(xlr[b], osh.at[dstv[b]],
                                      ssem[b]).wait()

            def compute_and_scatter(r, b):
                for g in range(kc // LNS):
                    gsl = pl.ds(g * LNS, LNS)
                    e16 = (r * kc + iota + g * LNS) * 4 + head
                    alph[gsl] = plsc.load_gather(pvb, [e16])
                    dstv[b][gsl] = dstb[pl.ds(r * kc + g * LNS, LNS)]
                xb = xlr[b]

                @pl.loop(0, kc, unroll=4)
                def _(e):
                    av = plsc.load_gather(alph,
                                          [jnp.zeros((LNS,), jnp.int32) + e])
                    for j2 in range(d // LNS):
                        jsl = pl.ds(j2 * LNS, LNS)
                        xb[e, jsl] = xb[e, jsl] * av

                pltpu.async_copy(xb, osh.at[dstv[b]], ssem[b], add=True)

            @pl.loop(0, nblk)
            def _(blk):
                eoff = s * ecw + blk * ib
                pltpu.sync_copy(src_hbm.at[pl.ds(eoff, ib)], gidxb)
                pltpu.sync_copy(dst_hbm.at[pl.ds(eoff, ib)], dstb)
                pltpu.sync_copy(pexp_hbm.at[pl.ds(eoff * 4, ib * 4)], pvb)
                for g in range(ib // LNS):
                    gsl = pl.ds(g * LNS, LNS)
                    gidxb[gsl] = gidxb[gsl] * 4 + head
                issue(0, 0)
                issue(1, 1)

                @pl.loop(0, blkc // nbuf)
                def _(kk):
                    for b in range(nbuf):
                        r = kk * nbuf + b
                        wait_rows(b)
                        nb = (b + 2) % nbuf

                        @pl.when(jnp.logical_and(r >= 2, r + 2 < blkc))
                        def _():
                            wait_scatter(nb)

                        @pl.when(r + 2 < blkc)
                        def _():
                            issue(r + 2, nb)
                        compute_and_scatter(r, b)

                for r in (blkc - 2, blkc - 1):
                    b = r % nbuf
                    wait_rows(b)
                    compute_and_scatter(r, b)

                for r in range(blkc - 4, blkc):
                    wait_scatter(r % nbuf)

            plsc.subcore_barrier()
            if hp == 0:
                @pl.when(c == 0)
                def _():
                    pltpu.sync_copy(osh.at[rsl], o0.at[rsl])

                @pl.when(c == 1)
                def _():
                    pltpu.sync_copy(osh.at[rsl], o2.at[rsl])
            else:
                @pl.when(c == 0)
                def _():
                    pltpu.sync_copy(osh.at[rsl], o1.at[rsl])

                @pl.when(c == 1)
                def _():
                    pltpu.sync_copy(osh.at[rsl], o3.at[rsl])
            plsc.subcore_barrier()

    return k(xl4, src, dsts, pexpf, z128)


def _post(oheads, dpart, inp, W_lin, b_lin, gat2d, ln_w, ln_b):
    n, d = inp.shape
    hd = W_lin.shape[1]
    BR = 1000

    def body(o0_ref, o1_ref, o2_ref, o3_ref, dp_ref, inp_ref,
             wl_ref, bl_ref, gb_ref, lw_ref, lb_ref, out_ref):
        dsum = dp_ref[...]
        acc = None
        wl = wl_ref[...]
        for h, oref in enumerate([o0_ref, o1_ref, o2_ref, o3_ref]):
            oh = oref[...] / dsum[:, h:h + 1] + gb_ref[h, :][None, :]
            t = lax.dot_general(oh, wl[:, h * d:(h + 1) * d], _DN,
                                preferred_element_type=jnp.float32)
            acc = t if acc is None else acc + t
        y = jnp.maximum(acc + bl_ref[...], 0.0)
        r = y + inp_ref[...]
        u = jnp.mean(r, axis=1, keepdims=True)
        var = jnp.mean((r - u) ** 2, axis=1, keepdims=True)
        out_ref[...] = lw_ref[...] * ((r - u) / jnp.sqrt(var + 1e-12)) + lb_ref[...]

    row_spec = pl.BlockSpec((BR, d), lambda i: (i, 0))
    full = lambda shape: pl.BlockSpec(shape, lambda i: (0,) * len(shape))
    return pl.pallas_call(
        body,
        grid=(n // BR,),
        in_specs=[row_spec, row_spec, row_spec, row_spec,
                  pl.BlockSpec((BR, 4), lambda i: (i, 0)),
                  row_spec,
                  full((d, hd)), full((1, d)), full((4, d)),
                  full((1, d)), full((1, d))],
        out_specs=row_spec,
        out_shape=jax.ShapeDtypeStruct((n, d), jnp.float32),
    )(*oheads, dpart, inp, W_lin, b_lin.reshape(1, d), gat2d,
      ln_w.reshape(1, d), ln_b.reshape(1, d))


def kernel(x, edge_index, W_AE, b_AE, Wl, bl, Wr, br, att, gat_bias,
           W_lin, b_lin, ln_w, ln_b):
    n, d = x.shape
    e = edge_index.shape[1]
    nheads = att.shape[0]

    inp, xl, xlb, xrb = _proj(x, W_AE, b_AE, Wl, bl, Wr, br)

    src0 = edge_index[0].astype(jnp.int32)
    dst0 = edge_index[1].astype(jnp.int32)
    loop = jnp.arange(n, dtype=jnp.int32)
    etot = e + n
    ce, kc = 64, 64
    ew = -(-etot // (NW * 128)) * 128
    e_pad = ew * NW
    pad = e_pad - etot
    src = jnp.concatenate([src0, loop, jnp.zeros((pad,), jnp.int32)])
    dstg = jnp.concatenate([dst0, loop, jnp.zeros((pad,), jnp.int32)])
    dsts = jnp.concatenate([dst0, loop, jnp.full((pad,), n, jnp.int32)])
    npad = -(-(n + 1) // 256) * 256

    hd2 = nheads * d // 2
    att_bf = att.reshape(-1).astype(jnp.bfloat16)
    attp = lax.bitcast_convert_type(
        jnp.stack([att_bf[:hd2], att_bf[hd2:]], axis=-1), jnp.int32)

    pexpf, dpart = _edge_logits(xlb, xrb, src, dstg, dsts, attp,
                                e_pad, ew, ce, npad)

    dfull = _dreduce(dpart)

    z128 = jnp.zeros((npad, d), jnp.float32)
    o = _aggregate(xl.reshape(n * nheads, d), src, dsts, pexpf, z128,
                   e_pad, npad, kc, d)

    return _post(list(o), dfull.reshape(npad, 4),
                 inp, W_lin, b_lin, gat_bias.reshape(nheads, d), ln_w, ln_b)
